# baseline (speedup 1.0000x reference)
import functools

import jax
import jax.numpy as jnp
from jax import lax
from jax.experimental import pallas as pl
from jax.experimental.pallas import tpu as pltpu
from jax.experimental.pallas import tpu_sc as plsc

HIDDEN = 64
ROWB = 2 * HIDDEN
CHUNK = 256
LANES = 16
WAVE = 8

_SC_PARAMS = dict(
    use_tc_tiling_on_sc=True,
    needs_layout_passes=False,
    disable_bounds_checks=True,
)


def _transpose_table(tabT, tail_rm):
    hid, v = tabT.shape
    tcols = v // 128
    vpad = -(-v // 128) * 128
    info = plsc.get_sparse_core_info()
    nw = info.num_cores * info.num_subcores
    u_per_w = -(-tcols // nw)
    if u_per_w % 2:
        u_per_w += 1
    n_pairs = u_per_w // 2
    mesh = plsc.VectorSubcoreMesh(core_axis_name="c", subcore_axis_name="s")

    @functools.partial(
        pl.kernel,
        mesh=mesh,
        out_type=jax.ShapeDtypeStruct((vpad // 2, ROWB), jnp.float32),
        scratch_types=[
            pltpu.VMEM((2, HIDDEN, 128), jnp.float32),
            pltpu.VMEM((2, HIDDEN, ROWB), jnp.float32),
            pltpu.SemaphoreType.DMA,
            pltpu.SemaphoreType.DMA,
        ],
        compiler_params=pltpu.CompilerParams(**_SC_PARAMS),
    )
    def tr(tabT_hbm, tail_hbm, dst_hbm, sstage, dstage, t0, t1):
        wid = lax.axis_index("s") * info.num_cores + lax.axis_index("c")
        u0 = wid * u_per_w
        tsem = (t0, t1)
        lanes = lax.iota(jnp.int32, LANES)

        def col0(u):
            c = jnp.minimum(u0 + u, tcols - 1) * 128
            return pl.multiple_of(c, 128)

        def fire(u, sl):
            pltpu.async_copy(
                tabT_hbm.at[:, pl.ds(col0(u), 128)], sstage.at[sl], tsem[sl]
            )

        def twait(u, sl):
            pltpu.make_async_copy(
                tabT_hbm.at[:, pl.ds(col0(u), 128)], sstage.at[sl], tsem[sl]
            ).wait()

        def flip_store(u, sl):
            src = sstage.at[sl]
            dstb = dstage.at[sl]

            def j_body(j, c2):
                vl0 = jnp.full((LANES,), 2 * j, jnp.int32)
                vl1 = vl0 + 1
                vals = [
                    plsc.load_gather(
                        src, [(16 * k % HIDDEN) + lanes, vl0 if k < 4 else vl1]
                    )
                    for k in range(8)
                ]
                for k in range(8):
                    dstb[j, pl.ds(16 * k, LANES)] = vals[k]
                return c2

            lax.fori_loop(0, HIDDEN, j_body, 0)
            j0 = pl.multiple_of(col0(u) // 2, HIDDEN)
            pltpu.sync_copy(dstb, dst_hbm.at[pl.ds(j0, HIDDEN), :])

        fire(0, 0)

        @pl.when(wid == 0)
        def _():
            tail_n = tail_hbm.shape[0]
            pltpu.sync_copy(tail_hbm, dstage.at[0].at[pl.ds(0, tail_n), :])
            tail_base = pl.multiple_of(tcols * 128 // 2, 8)
            pltpu.sync_copy(
                dstage.at[0].at[pl.ds(0, tail_n), :],
                dst_hbm.at[pl.ds(tail_base, tail_n), :],
            )

        def pair_body(g, carry):
            ua = 2 * g
            twait(ua, 0)
            fire(ua + 1, 1)
            flip_store(ua, 0)
            twait(ua + 1, 1)
            fire(jnp.minimum(ua + 2, u_per_w - 1), 0)
            flip_store(ua + 1, 1)
            return carry

        lax.fori_loop(0, n_pairs, pair_body, 0)
        twait(u_per_w - 1, 0)

    return tr(tabT, tail_rm)


def _embed(idx, tpacked, s_count, b_count):
    n = idx.shape[0]
    info = plsc.get_sparse_core_info()
    nw = info.num_cores * info.num_subcores
    blocks_per_s = b_count // CHUNK
    u_per_w = (n // CHUNK) // nw
    n_pairs = u_per_w // 2
    mesh = plsc.VectorSubcoreMesh(core_axis_name="c", subcore_axis_name="s")

    @functools.partial(
        pl.kernel,
        mesh=mesh,
        out_type=jax.ShapeDtypeStruct((s_count, HIDDEN, b_count), jnp.float32),
        scratch_types=[
            pltpu.VMEM((u_per_w * CHUNK,), jnp.int32),
            pltpu.VMEM((u_per_w * CHUNK,), jnp.int32),
            pltpu.VMEM((2, CHUNK, ROWB), jnp.float32),
            pltpu.VMEM((2, HIDDEN, CHUNK), jnp.float32),
            pltpu.SemaphoreType.DMA,
            pltpu.SemaphoreType.DMA,
        ],
        compiler_params=pltpu.CompilerParams(**_SC_PARAMS),
    )
    def emb(idx_hbm, tab_hbm, out_hbm, idx_all, vrow_all, rows_v, stage_v, g0, g1):
        wid = lax.axis_index("s") * info.num_cores + lax.axis_index("c")
        u0 = wid * u_per_w
        tok0 = pl.multiple_of(u0 * CHUNK, CHUNK)
        pltpu.sync_copy(idx_hbm.at[pl.ds(tok0, u_per_w * CHUNK)], idx_all)
        gsem = (g0, g1)
        lanes = lax.iota(jnp.int32, LANES)

        def prow_body(m, c2):
            vv = idx_all[pl.ds(m * LANES, LANES)]
            vrow_all[pl.ds(m * LANES, LANES)] = lax.shift_right_logical(vv, 1)
            return c2

        lax.fori_loop(0, u_per_w * CHUNK // LANES, prow_body, 0)

        def vrow_slice(ul):
            o = pl.multiple_of(ul * CHUNK, CHUNK)
            return vrow_all.at[pl.ds(o, CHUNK)]

        def fire(ul, sl):
            pltpu.async_copy(tab_hbm.at[vrow_slice(ul)], rows_v.at[sl], gsem[sl])

        def gwait(ul, sl):
            pltpu.make_async_copy(
                tab_hbm.at[vrow_slice(ul)], rows_v.at[sl], gsem[sl]
            ).wait()

        def transpose_store(ul, sl):
            rows = rows_v.at[sl]
            stage = stage_v.at[sl]

            def m_body(m, c2):
                tok = m * LANES + lanes
                o = ul * CHUNK + m * LANES
                vv = idx_all[pl.ds(o, LANES)]
                half = (vv & 1) * HIDDEN
                for w in range(0, HIDDEN, WAVE):
                    vals = [
                        plsc.load_gather(rows, [tok, half + (w + k)])
                        for k in range(WAVE)
                    ]
                    for k in range(WAVE):
                        stage[w + k, pl.ds(m * LANES, LANES)] = vals[k]
                return c2

            lax.fori_loop(0, CHUNK // LANES, m_body, 0)
            uid = u0 + ul
            s = uid // blocks_per_s
            b0 = pl.multiple_of((uid % blocks_per_s) * CHUNK, CHUNK)
            pltpu.sync_copy(stage, out_hbm.at[s, :, pl.ds(b0, CHUNK)])

        fire(0, 0)

        def pair_body(g, carry):
            ua = 2 * g
            gwait(ua, 0)
            fire(ua + 1, 1)
            transpose_store(ua, 0)
            gwait(ua + 1, 1)
            fire(jnp.minimum(ua + 2, u_per_w - 1), 0)
            transpose_store(ua + 1, 1)
            return carry

        lax.fori_loop(0, n_pairs, pair_body, 0)
        gwait(u_per_w - 1, 0)

    return emb(idx, tpacked)


def kernel(tokens, embedding_table):
    b, s = tokens.shape
    v = embedding_table.shape[0]
    idx = tokens.T.reshape(b * s)
    vfull = v // 128 * 128
    tail_rm = embedding_table[vfull:].reshape(-1, ROWB)
    tpacked = _transpose_table(embedding_table.T, tail_rm)
    out3 = _embed(idx, tpacked, s, b)
    return (tokens, out3.transpose(2, 0, 1))


# speedup vs baseline: 1.6647x; 1.6647x over previous
import functools

import jax
import jax.numpy as jnp
from jax import lax
from jax.experimental import pallas as pl
from jax.experimental.pallas import tpu as pltpu
from jax.experimental.pallas import tpu_sc as plsc

HIDDEN = 64
CHUNK = 1600


def _embed(idx, table):
    n = idx.shape[0]
    info = plsc.get_sparse_core_info()
    nw = info.num_cores * info.num_subcores
    n_per_w = n // nw
    n_chunks = n_per_w // CHUNK
    mesh = plsc.VectorSubcoreMesh(core_axis_name="c", subcore_axis_name="s")

    @functools.partial(
        pl.kernel,
        mesh=mesh,
        out_type=jax.ShapeDtypeStruct((n, HIDDEN), jnp.float32),
        scratch_types=[
            pltpu.VMEM((CHUNK,), jnp.int32),
            pltpu.VMEM((CHUNK, HIDDEN), jnp.float32),
            pltpu.SemaphoreType.DMA,
        ],
        compiler_params=pltpu.CompilerParams(use_tc_tiling_on_sc=False),
    )
    def emb(idx_hbm, table_hbm, out_hbm, idx_v, rows_v, sem):
        wid = lax.axis_index("s") * info.num_cores + lax.axis_index("c")
        base = wid * n_per_w

        def body(i, carry):
            off = base + i * CHUNK
            pltpu.sync_copy(idx_hbm.at[pl.ds(off, CHUNK)], idx_v)
            pltpu.async_copy(table_hbm.at[idx_v], rows_v, sem).wait()
            pltpu.sync_copy(rows_v, out_hbm.at[pl.ds(off, CHUNK)])
            return carry

        lax.fori_loop(0, n_chunks, body, 0)

    return emb(idx, table)


def kernel(tokens, embedding_table):
    b, s = tokens.shape
    idx = tokens.reshape(b * s).astype(jnp.int32)
    out = _embed(idx, embedding_table)
    return (tokens, out.reshape(b, s, HIDDEN))
